# baseline (speedup 1.0000x reference)
import functools

import jax
import jax.numpy as jnp
from jax import lax
from jax.experimental import pallas as pl
from jax.experimental.pallas import tpu as pltpu
from jax.experimental.pallas import tpu_sc as plsc

NUM_LOC = 10000
NUM_CAT = 512
NUM_EDGES = 320000
LOC_DIM = 128
CAT_DIM = 128

NC = 2
NS = 16
NW = NC * NS
CHUNK = 125
EDGES_PER_W = NUM_EDGES // NW
NCHUNK = EDGES_PER_W // CHUNK
NBUF = 4

_sc_mesh = plsc.VectorSubcoreMesh(core_axis_name="c", subcore_axis_name="s")


@functools.partial(
    pl.kernel,
    out_type=jax.ShapeDtypeStruct((NC, NUM_CAT, LOC_DIM), jnp.float32),
    mesh=_sc_mesh,
    scratch_types=[
        pltpu.VMEM_SHARED((NUM_CAT, LOC_DIM), jnp.float32),
        pltpu.VMEM((NCHUNK, CHUNK), jnp.int32),
        pltpu.VMEM((NCHUNK, CHUNK), jnp.int32),
        [pltpu.VMEM((CHUNK, LOC_DIM), jnp.float32) for _ in range(NBUF)],
        [pltpu.SemaphoreType.DMA for _ in range(NBUF)],
    ],
)
def _sc_agg(loc_hbm, src_hbm, dst_hbm, zeros_hbm, out_hbm,
            agg_sh, src_v, dst_v, bufs, gsems):
    c = lax.axis_index("c")
    s = lax.axis_index("s")
    wid = c * NS + s

    @pl.when(s == 0)
    def _():
        pltpu.sync_copy(zeros_hbm, agg_sh)

    pltpu.sync_copy(src_hbm.at[wid], src_v)
    pltpu.sync_copy(dst_hbm.at[wid], dst_v)
    plsc.subcore_barrier()

    for p in range(NBUF - 1):
        pltpu.async_copy(loc_hbm.at[src_v.at[p]], bufs[p], gsems[p])

    def body(j, _):
        for p in range(NBUF):
            i = j * NBUF + p
            pltpu.make_async_copy(loc_hbm.at[src_v.at[i]], bufs[p],
                                  gsems[p]).wait()
            pltpu.sync_copy(bufs[p], agg_sh.at[dst_v.at[i]], add=True)

            @pl.when(i + NBUF - 1 < NCHUNK)
            def _():
                q = (p + NBUF - 1) % NBUF
                pltpu.async_copy(loc_hbm.at[src_v.at[i + NBUF - 1]],
                                 bufs[q], gsems[q])
        return 0

    lax.fori_loop(0, NCHUNK // NBUF, body, 0)

    plsc.subcore_barrier()
    rows = NUM_CAT // NS
    pltpu.sync_copy(agg_sh.at[pl.ds(s * rows, rows)],
                    out_hbm.at[c, pl.ds(s * rows, rows)])


def _tc_linear_body(p_ref, w_ref, b_ref, o_ref):
    acc = p_ref[0] + p_ref[1]
    x = lax.dot_general(acc, w_ref[...], (((1,), (1,)), ((), ())),
                        preferred_element_type=jnp.float32)
    x = x + b_ref[...]
    o_ref[...] = jnp.where(x > 0, x, 0.2 * x)


_tc_linear = pl.pallas_call(
    _tc_linear_body,
    out_shape=jax.ShapeDtypeStruct((NUM_CAT, CAT_DIM), jnp.float32),
)


def kernel(loc_emb, edge_src, edge_dst, W, b):
    src3 = edge_src.astype(jnp.int32).reshape(NW, NCHUNK, CHUNK)
    dst3 = edge_dst.astype(jnp.int32).reshape(NW, NCHUNK, CHUNK)
    zeros = jnp.zeros((NUM_CAT, LOC_DIM), jnp.float32)
    partials = _sc_agg(loc_emb, src3, dst3, zeros)
    return _tc_linear(partials, W, b.reshape(1, CAT_DIM))


# speedup vs baseline: 1.0005x; 1.0005x over previous
import functools

import jax
import jax.numpy as jnp
from jax import lax
from jax.experimental import pallas as pl
from jax.experimental.pallas import tpu as pltpu
from jax.experimental.pallas import tpu_sc as plsc

NUM_LOC = 10000
NUM_CAT = 512
NUM_EDGES = 320000
LOC_DIM = 128
CAT_DIM = 128

NC = 2
NS = 16
NW = NC * NS
CHUNK = 125
EDGES_PER_W = NUM_EDGES // NW
NCHUNK = EDGES_PER_W // CHUNK
NBUF = 6
ZROWS = NUM_CAT // NS

_sc_mesh = plsc.VectorSubcoreMesh(core_axis_name="c", subcore_axis_name="s")


@functools.partial(
    pl.kernel,
    out_type=jax.ShapeDtypeStruct((NC, NUM_CAT, LOC_DIM), jnp.float32),
    mesh=_sc_mesh,
    scratch_types=[
        pltpu.VMEM_SHARED((NUM_CAT, LOC_DIM), jnp.float32),
        pltpu.VMEM((NCHUNK, CHUNK), jnp.int32),
        pltpu.VMEM((NCHUNK, CHUNK), jnp.int32),
        [pltpu.VMEM((CHUNK, LOC_DIM), jnp.float32) for _ in range(NBUF)],
        [pltpu.SemaphoreType.DMA for _ in range(NBUF)],
        pltpu.SemaphoreType.DMA,
        pltpu.SemaphoreType.DMA,
    ],
)
def _sc_agg(loc_hbm, src_hbm, dst_hbm, zeros_hbm, out_hbm,
            agg_sh, src_v, dst_v, bufs, gsems, sem_s, sem_d):
    c = lax.axis_index("c")
    s = lax.axis_index("s")
    wid = c * NS + s

    pltpu.async_copy(src_hbm.at[wid], src_v, sem_s)
    pltpu.async_copy(dst_hbm.at[wid], dst_v, sem_d)
    pltpu.make_async_copy(src_hbm.at[wid], src_v, sem_s).wait()

    for p in range(NBUF - 1):
        pltpu.async_copy(loc_hbm.at[src_v.at[p]], bufs[p], gsems[p])

    pltpu.sync_copy(zeros_hbm, agg_sh.at[pl.ds(s * ZROWS, ZROWS)])
    pltpu.make_async_copy(dst_hbm.at[wid], dst_v, sem_d).wait()
    plsc.subcore_barrier()


    def body(j, _):
        for p in range(NBUF):
            i = j * NBUF + p
            pltpu.make_async_copy(loc_hbm.at[src_v.at[i]], bufs[p],
                                  gsems[p]).wait()
            pltpu.sync_copy(bufs[p], agg_sh.at[dst_v.at[i]], add=True)

            @pl.when(i + NBUF - 1 < NCHUNK)
            def _():
                q = (p + NBUF - 1) % NBUF
                pltpu.async_copy(loc_hbm.at[src_v.at[i + NBUF - 1]],
                                 bufs[q], gsems[q])
        return 0

    lax.fori_loop(0, NCHUNK // NBUF, body, 0)
    for i in range(NCHUNK - NCHUNK % NBUF, NCHUNK):
        p = i % NBUF
        pltpu.make_async_copy(loc_hbm.at[src_v.at[i]], bufs[p],
                              gsems[p]).wait()
        pltpu.sync_copy(bufs[p], agg_sh.at[dst_v.at[i]], add=True)

    plsc.subcore_barrier()
    pltpu.sync_copy(agg_sh.at[pl.ds(s * ZROWS, ZROWS)],
                    out_hbm.at[c, pl.ds(s * ZROWS, ZROWS)])


def _tc_linear_body(p_ref, w_ref, b_ref, o_ref):
    acc = p_ref[0] + p_ref[1]
    x = lax.dot_general(acc, w_ref[...], (((1,), (1,)), ((), ())),
                        preferred_element_type=jnp.float32)
    x = x + b_ref[...]
    o_ref[...] = jnp.where(x > 0, x, 0.2 * x)


_tc_linear = pl.pallas_call(
    _tc_linear_body,
    out_shape=jax.ShapeDtypeStruct((NUM_CAT, CAT_DIM), jnp.float32),
)


def kernel(loc_emb, edge_src, edge_dst, W, b):
    src3 = edge_src.astype(jnp.int32).reshape(NW, NCHUNK, CHUNK)
    dst3 = edge_dst.astype(jnp.int32).reshape(NW, NCHUNK, CHUNK)
    zeros = jnp.zeros((ZROWS, LOC_DIM), jnp.float32)
    partials = _sc_agg(loc_emb, src3, dst3, zeros)
    return _tc_linear(partials, W, b.reshape(1, CAT_DIM))
